# baseline (speedup 1.0000x reference)
import functools

import jax
import jax.numpy as jnp
from jax.experimental import pallas as pl
from jax.experimental.pallas import tpu as pltpu

_N = 4096
_IB = 512
_JB = 1024
_B = 256


def _gnn_body(nj, transpose_out, a_ref, xj_ref, xi_ref, wr_ref, ws_ref,
              b_ref, o_ref):
    j = pl.program_id(1)
    a_pos = jnp.maximum(a_ref[...], 0.0)
    y = jnp.dot(xj_ref[...], wr_ref[...], preferred_element_type=jnp.float32, precision=jax.lax.Precision.HIGHEST)

    @pl.when(j == 0)
    def _():
        if transpose_out:
            base = jax.lax.dot_general(
                ws_ref[...], xi_ref[...], (((0,), (1,)), ((), ())),
                preferred_element_type=jnp.float32, precision=jax.lax.Precision.HIGHEST)
        else:
            base = jnp.dot(xi_ref[...], ws_ref[...],
                           preferred_element_type=jnp.float32, precision=jax.lax.Precision.HIGHEST)
        o_ref[...] = base + b_ref[...]

    if transpose_out:
        part = jax.lax.dot_general(y, a_pos, (((0,), (0,)), ((), ())),
                                   preferred_element_type=jnp.float32, precision=jax.lax.Precision.HIGHEST)
    else:
        part = jax.lax.dot_general(a_pos, y, (((0,), (0,)), ((), ())),
                                   preferred_element_type=jnp.float32, precision=jax.lax.Precision.HIGHEST)
    o_ref[...] += part

    @pl.when(j == nj - 1)
    def _():
        o_ref[...] = jnp.maximum(o_ref[...], 0.0)


def _gnn_layer(x, a, Wr, Ws, b, transpose_out):
    f_in = x.shape[1]
    f_out = Wr.shape[1]
    ni, nj = _N // _IB, _N // _JB
    b2 = b[:, None] if transpose_out else b[None, :]
    if transpose_out:
        out_shape = jax.ShapeDtypeStruct((f_out, _N), jnp.float32)
        out_spec = pl.BlockSpec((f_out, _IB), lambda i, j: (0, i))
    else:
        out_shape = jax.ShapeDtypeStruct((_N, f_out), jnp.float32)
        out_spec = pl.BlockSpec((_IB, f_out), lambda i, j: (i, 0))
    return pl.pallas_call(
        functools.partial(_gnn_body, nj, transpose_out),
        grid=(ni, nj),
        in_specs=[
            pl.BlockSpec((_JB, _IB), lambda i, j: (j, i)),
            pl.BlockSpec((_JB, f_in), lambda i, j: (j, 0)),
            pl.BlockSpec((_IB, f_in), lambda i, j: (i, 0)),
            pl.BlockSpec((f_in, f_out), lambda i, j: (0, 0)),
            pl.BlockSpec((f_in, f_out), lambda i, j: (0, 0)),
            pl.BlockSpec(b2.shape, lambda i, j: (0, 0)),
        ],
        out_specs=out_spec,
        out_shape=out_shape,
        compiler_params=pltpu.CompilerParams(
            dimension_semantics=("parallel", "arbitrary")),
    )(a, x, x, Wr, Ws, b2)


def _up(c):
    prev = jnp.concatenate([c[:, :1], c[:, :-1]], axis=1)
    nxt = jnp.concatenate([c[:, 1:], c[:, -1:]], axis=1)
    e = 0.75 * c + 0.25 * prev
    o = 0.75 * c + 0.25 * nxt
    cc, ll, bb = c.shape
    return jnp.stack([e, o], axis=2).reshape(cc, 2 * ll, bb)


def _cnn_body(m1_ref, cb1_ref, m2_ref, cb2_ref, m3_ref, cb3_ref, m4_ref,
              cb4_ref, m5_ref, cb5_ref, h_ref, o_ref):
    x0 = h_ref[...]
    z1 = jnp.zeros((1, _B), jnp.float32)
    xp = jnp.concatenate([z1, x0, z1], axis=0)
    c = jnp.dot(m1_ref[...], xp, preferred_element_type=jnp.float32, precision=jax.lax.Precision.HIGHEST)
    c = c.reshape(32, 16, _B) + cb1_ref[...][:, :, None]
    u = jnp.maximum(_up(c), 0.0)

    for m_ref, cb_ref in ((m2_ref, cb2_ref), (m3_ref, cb3_ref),
                          (m4_ref, cb4_ref)):
        ll = u.shape[1]
        zp = jnp.zeros((32, 2, _B), jnp.float32)
        hp = jnp.concatenate([zp, u, zp], axis=1)
        parts = []
        for s0 in range(0, ll, 8):
            w = hp[:, s0:s0 + 12, :].reshape(384, _B)
            parts.append(
                jnp.dot(m_ref[...], w,
                        preferred_element_type=jnp.float32, precision=jax.lax.Precision.HIGHEST).reshape(32, 8, _B))
        c = jnp.concatenate(parts, axis=1) + cb_ref[...][:, :, None]
        u = jnp.maximum(_up(c), 0.0)

    hp = jnp.concatenate([u, jnp.zeros((32, 4, _B), jnp.float32)], axis=1)
    rows = []
    for s0 in range(0, 256, 8):
        w = hp[:, s0:s0 + 12, :].reshape(384, _B)
        rows.append(jnp.dot(m5_ref[...], w,
                            preferred_element_type=jnp.float32, precision=jax.lax.Precision.HIGHEST))
    c5 = jnp.concatenate(rows, axis=0)
    o_ref[...] = c5[1:251, :] + cb5_ref[0, 0]


def _mk_w1(cw1):
    wt = cw1[:, 0, :].T
    arr = jnp.zeros((18, 32, 16), jnp.float32)
    for d in range(16):
        arr = arr.at[d:d + 3, :, d].set(wt)
    return arr.reshape(18, 512).T


def _mk_stage(cw):
    wt = jnp.transpose(cw, (1, 2, 0))
    arr = jnp.zeros((32, 12, 32, 8), jnp.float32)
    for d in range(8):
        arr = arr.at[:, d:d + 5, :, d].set(wt)
    return arr.reshape(384, 256).T


def _mk_w5(cw5):
    arr = jnp.zeros((32, 12, 8), jnp.float32)
    for d in range(8):
        arr = arr.at[:, d:d + 5, d].set(cw5[0])
    return arr.reshape(384, 8).T


def kernel(z, a, Wr0, Ws0, b0, Wr1, Ws1, b1, Wr2, Ws2, b2, cw1, cb1, cw2,
           cb2, cw3, cb3, cw4, cb4, cw5, cb5, logvar_x):
    x1 = _gnn_layer(z, a, Wr0, Ws0, b0, False)
    x2 = _gnn_layer(x1, a, Wr1, Ws1, b1, False)
    hT = _gnn_layer(x2, a, Wr2, Ws2, b2, True)

    m1 = _mk_w1(cw1)
    m2 = _mk_stage(cw2)
    m3 = _mk_stage(cw3)
    m4 = _mk_stage(cw4)
    m5 = _mk_w5(cw5)
    full = lambda shape: pl.BlockSpec(shape, lambda i: (0, 0))
    out2d = pl.pallas_call(
        _cnn_body,
        grid=(_N // _B,),
        in_specs=[
            full((512, 18)), full((32, 1)),
            full((256, 384)), full((32, 1)),
            full((256, 384)), full((32, 1)),
            full((256, 384)), full((32, 1)),
            full((8, 384)), full((1, 1)),
            pl.BlockSpec((16, _B), lambda i: (0, i)),
        ],
        out_specs=pl.BlockSpec((250, _B), lambda i: (0, i)),
        out_shape=jax.ShapeDtypeStruct((250, _N), jnp.float32),
        compiler_params=pltpu.CompilerParams(
            dimension_semantics=("parallel",)),
    )(m1, cb1[:, None], m2, cb2[:, None], m3, cb3[:, None], m4,
      cb4[:, None], m5, cb5[:, None], hT)

    x = out2d.T.reshape(1, _N, 250)
    return x, jnp.exp(0.5 * logvar_x)


# speedup vs baseline: 1.0674x; 1.0674x over previous
import functools

import jax
import jax.numpy as jnp
from jax.experimental import pallas as pl
from jax.experimental.pallas import tpu as pltpu

_N = 4096
_IB = 512
_JB = 1024
_B = 256


def _split(x):
    hi = x.astype(jnp.bfloat16)
    lo = (x - hi.astype(jnp.float32)).astype(jnp.bfloat16)
    return hi, lo


def _dot3s(ah, al, bh, bl, dims):
    d = lambda u, v: jax.lax.dot_general(u, v, (dims, ((), ())),
                                         preferred_element_type=jnp.float32)
    return d(ah, bh) + d(ah, bl) + d(al, bh)


def _dot3(a, b, dims=((1,), (0,))):
    ah, al = _split(a)
    bh, bl = _split(b)
    return _dot3s(ah, al, bh, bl, dims)


def _gnn_body(nj, transpose_out, a_ref, xj_ref, xi_ref, wr_ref, ws_ref,
              b_ref, o_ref):
    j = pl.program_id(1)
    y = _dot3(xj_ref[...], wr_ref[...])

    @pl.when(j == 0)
    def _():
        if transpose_out:
            base = _dot3(ws_ref[...], xi_ref[...], dims=((0,), (1,)))
        else:
            base = _dot3(xi_ref[...], ws_ref[...])
        o_ref[...] = base + b_ref[...]

    if transpose_out:
        part = _dot3(y, a_ref[...], dims=((0,), (0,)))
    else:
        part = _dot3(a_ref[...], y, dims=((0,), (0,)))
    o_ref[...] += part

    @pl.when(j == nj - 1)
    def _():
        o_ref[...] = jnp.maximum(o_ref[...], 0.0)


def _gnn_layer(x, a, Wr, Ws, b, transpose_out):
    f_in = x.shape[1]
    f_out = Wr.shape[1]
    ni, nj = _N // _IB, _N // _JB
    b2 = b[:, None] if transpose_out else b[None, :]
    if transpose_out:
        out_shape = jax.ShapeDtypeStruct((f_out, _N), jnp.float32)
        out_spec = pl.BlockSpec((f_out, _IB), lambda i, j: (0, i))
    else:
        out_shape = jax.ShapeDtypeStruct((_N, f_out), jnp.float32)
        out_spec = pl.BlockSpec((_IB, f_out), lambda i, j: (i, 0))
    return pl.pallas_call(
        functools.partial(_gnn_body, nj, transpose_out),
        grid=(ni, nj),
        in_specs=[
            pl.BlockSpec((_JB, _IB), lambda i, j: (j, i)),
            pl.BlockSpec((_JB, f_in), lambda i, j: (j, 0)),
            pl.BlockSpec((_IB, f_in), lambda i, j: (i, 0)),
            pl.BlockSpec((f_in, f_out), lambda i, j: (0, 0)),
            pl.BlockSpec((f_in, f_out), lambda i, j: (0, 0)),
            pl.BlockSpec(b2.shape, lambda i, j: (0, 0)),
        ],
        out_specs=out_spec,
        out_shape=out_shape,
        compiler_params=pltpu.CompilerParams(
            dimension_semantics=("parallel", "arbitrary")),
    )(a, x, x, Wr, Ws, b2)


def _up(c):
    prev = jnp.concatenate([c[:, :1], c[:, :-1]], axis=1)
    nxt = jnp.concatenate([c[:, 1:], c[:, -1:]], axis=1)
    e = 0.75 * c + 0.25 * prev
    o = 0.75 * c + 0.25 * nxt
    cc, ll, bb = c.shape
    return jnp.stack([e, o], axis=2).reshape(cc, 2 * ll, bb)


def _pad_split(u, left, right):
    uh, ul = _split(u)
    zl = [jnp.zeros((u.shape[0], left, u.shape[2]), jnp.bfloat16)] if left else []
    zr = [jnp.zeros((u.shape[0], right, u.shape[2]), jnp.bfloat16)] if right else []
    hp_h = jnp.concatenate(zl + [uh] + zr, axis=1)
    hp_l = jnp.concatenate(zl + [ul] + zr, axis=1)
    return hp_h, hp_l


def _cnn_body(m1_ref, cb1_ref, m2_ref, cb2_ref, m3_ref, cb3_ref, m4_ref,
              cb4_ref, m5_ref, cb5_ref, h_ref, o_ref):
    x0 = h_ref[...]
    z1 = jnp.zeros((1, _B), jnp.float32)
    xp = jnp.concatenate([z1, x0, z1], axis=0)
    c = _dot3(m1_ref[...], xp)
    c = c.reshape(32, 16, _B) + cb1_ref[...][:, :, None]
    u = jnp.maximum(_up(c), 0.0)

    for m_ref, cb_ref in ((m2_ref, cb2_ref), (m3_ref, cb3_ref),
                          (m4_ref, cb4_ref)):
        ll = u.shape[1]
        mh, ml = _split(m_ref[...])
        hp_h, hp_l = _pad_split(u, 2, 2)
        parts = []
        for s0 in range(0, ll, 8):
            wh = hp_h[:, s0:s0 + 12, :].reshape(384, _B)
            wl = hp_l[:, s0:s0 + 12, :].reshape(384, _B)
            r = _dot3s(mh, ml, wh, wl, ((1,), (0,)))
            parts.append(r.reshape(32, 8, _B))
        c = jnp.concatenate(parts, axis=1) + cb_ref[...][:, :, None]
        u = jnp.maximum(_up(c), 0.0)

    mh, ml = _split(m5_ref[...])
    hp_h, hp_l = _pad_split(u, 0, 4)
    rows = []
    for s0 in range(0, 256, 8):
        wh = hp_h[:, s0:s0 + 12, :].reshape(384, _B)
        wl = hp_l[:, s0:s0 + 12, :].reshape(384, _B)
        rows.append(_dot3s(mh, ml, wh, wl, ((1,), (0,))))
    c5 = jnp.concatenate(rows, axis=0)
    o_ref[...] = c5[1:251, :] + cb5_ref[0, 0]


def _mk_w1(cw1):
    wt = cw1[:, 0, :].T
    arr = jnp.zeros((18, 32, 16), jnp.float32)
    for d in range(16):
        arr = arr.at[d:d + 3, :, d].set(wt)
    return arr.reshape(18, 512).T


def _mk_stage(cw):
    wt = jnp.transpose(cw, (1, 2, 0))
    arr = jnp.zeros((32, 12, 32, 8), jnp.float32)
    for d in range(8):
        arr = arr.at[:, d:d + 5, :, d].set(wt)
    return arr.reshape(384, 256).T


def _mk_w5(cw5):
    arr = jnp.zeros((32, 12, 8), jnp.float32)
    for d in range(8):
        arr = arr.at[:, d:d + 5, d].set(cw5[0])
    return arr.reshape(384, 8).T


def kernel(z, a, Wr0, Ws0, b0, Wr1, Ws1, b1, Wr2, Ws2, b2, cw1, cb1, cw2,
           cb2, cw3, cb3, cw4, cb4, cw5, cb5, logvar_x):
    x1 = _gnn_layer(z, a, Wr0, Ws0, b0, False)
    x2 = _gnn_layer(x1, a, Wr1, Ws1, b1, False)
    hT = _gnn_layer(x2, a, Wr2, Ws2, b2, True)

    m1 = _mk_w1(cw1)
    m2 = _mk_stage(cw2)
    m3 = _mk_stage(cw3)
    m4 = _mk_stage(cw4)
    m5 = _mk_w5(cw5)
    full = lambda shape: pl.BlockSpec(shape, lambda i: (0, 0))
    out2d = pl.pallas_call(
        _cnn_body,
        grid=(_N // _B,),
        in_specs=[
            full((512, 18)), full((32, 1)),
            full((256, 384)), full((32, 1)),
            full((256, 384)), full((32, 1)),
            full((256, 384)), full((32, 1)),
            full((8, 384)), full((1, 1)),
            pl.BlockSpec((16, _B), lambda i: (0, i)),
        ],
        out_specs=pl.BlockSpec((250, _B), lambda i: (0, i)),
        out_shape=jax.ShapeDtypeStruct((250, _N), jnp.float32),
        compiler_params=pltpu.CompilerParams(
            dimension_semantics=("parallel",)),
    )(m1, cb1[:, None], m2, cb2[:, None], m3, cb3[:, None], m4,
      cb4[:, None], m5, cb5[:, None], hT)

    x = out2d.T.reshape(1, _N, 250)
    return x, jnp.exp(0.5 * logvar_x)


# speedup vs baseline: 1.4499x; 1.3583x over previous
import functools

import jax
import jax.numpy as jnp
from jax.experimental import pallas as pl
from jax.experimental.pallas import tpu as pltpu

_N = 4096
_IB = 512
_JB = 1024
_B = 256


def _split(x):
    hi = x.astype(jnp.bfloat16)
    lo = (x - hi.astype(jnp.float32)).astype(jnp.bfloat16)
    return hi, lo


def _dot3s(ah, al, bh, bl, dims):
    d = lambda u, v: jax.lax.dot_general(u, v, (dims, ((), ())),
                                         preferred_element_type=jnp.float32)
    return d(ah, bh) + d(ah, bl) + d(al, bh)


def _dot3(a, b, dims=((1,), (0,))):
    ah, al = _split(a)
    bh, bl = _split(b)
    return _dot3s(ah, al, bh, bl, dims)


def _gnn_body(nj, transpose_out, a_ref, xj_ref, xi_ref, wr_ref, ws_ref,
              b_ref, o_ref):
    j = pl.program_id(1)
    y = _dot3(xj_ref[...], wr_ref[...])

    @pl.when(j == 0)
    def _():
        if transpose_out:
            base = _dot3(ws_ref[...], xi_ref[...], dims=((0,), (1,)))
        else:
            base = _dot3(xi_ref[...], ws_ref[...])
        o_ref[...] = base + b_ref[...]

    ah, al = _split(a_ref[...])
    yh, yl = _split(y)
    f = y.shape[1]
    d0 = lambda u, v: jax.lax.dot_general(
        u, v, ((((0,), (0,))), ((), ())),
        preferred_element_type=jnp.float32)
    if transpose_out:
        p = d0(jnp.concatenate([yh, yl], axis=1), ah)
        part = p[:f] + p[f:] + d0(yh, al)
    else:
        p = d0(ah, jnp.concatenate([yh, yl], axis=1))
        part = p[:, :f] + p[:, f:] + d0(al, yh)
    o_ref[...] += part

    @pl.when(j == nj - 1)
    def _():
        o_ref[...] = jnp.maximum(o_ref[...], 0.0)


def _gnn_layer(x, a, Wr, Ws, b, transpose_out):
    f_in = x.shape[1]
    f_out = Wr.shape[1]
    ni, nj = _N // _IB, _N // _JB
    b2 = b[:, None] if transpose_out else b[None, :]
    if transpose_out:
        out_shape = jax.ShapeDtypeStruct((f_out, _N), jnp.float32)
        out_spec = pl.BlockSpec((f_out, _IB), lambda i, j: (0, i))
    else:
        out_shape = jax.ShapeDtypeStruct((_N, f_out), jnp.float32)
        out_spec = pl.BlockSpec((_IB, f_out), lambda i, j: (i, 0))
    return pl.pallas_call(
        functools.partial(_gnn_body, nj, transpose_out),
        grid=(ni, nj),
        in_specs=[
            pl.BlockSpec((_JB, _IB), lambda i, j: (j, i)),
            pl.BlockSpec((_JB, f_in), lambda i, j: (j, 0)),
            pl.BlockSpec((_IB, f_in), lambda i, j: (i, 0)),
            pl.BlockSpec((f_in, f_out), lambda i, j: (0, 0)),
            pl.BlockSpec((f_in, f_out), lambda i, j: (0, 0)),
            pl.BlockSpec(b2.shape, lambda i, j: (0, 0)),
        ],
        out_specs=out_spec,
        out_shape=out_shape,
        compiler_params=pltpu.CompilerParams(
            dimension_semantics=("parallel", "arbitrary")),
    )(a, x, x, Wr, Ws, b2)


def _up(c):
    prev = jnp.concatenate([c[:, :1], c[:, :-1]], axis=1)
    nxt = jnp.concatenate([c[:, 1:], c[:, -1:]], axis=1)
    e = 0.75 * c + 0.25 * prev
    o = 0.75 * c + 0.25 * nxt
    cc, ll, bb = c.shape
    return jnp.stack([e, o], axis=2).reshape(cc, 2 * ll, bb)


def _cnn_body(m1_ref, cb1_ref, m2_ref, cb2_ref, m3_ref, cb3_ref, m4_ref,
              cb4_ref, m5_ref, cb5_ref, h_ref, o_ref):
    dot = lambda u, v: jax.lax.dot_general(
        u, v, ((((1,), (0,))), ((), ())),
        preferred_element_type=jnp.float32)
    x0 = h_ref[...]
    z1 = jnp.zeros((1, _B), jnp.float32)
    xp = jnp.concatenate([z1, x0, z1], axis=0)
    c = dot(m1_ref[...], xp)
    c = c.reshape(32, 16, _B) + cb1_ref[...][:, :, None]
    u = jnp.maximum(_up(c), 0.0)

    for m_ref, cb_ref in ((m2_ref, cb2_ref), (m3_ref, cb3_ref),
                          (m4_ref, cb4_ref)):
        ll = u.shape[1]
        zp = jnp.zeros((32, 2, _B), jnp.float32)
        hp = jnp.concatenate([zp, u, zp], axis=1)
        parts = []
        for s0 in range(0, ll, 16):
            w = hp[:, s0:s0 + 20, :].reshape(640, _B)
            parts.append(dot(m_ref[...], w).reshape(32, 16, _B))
        c = jnp.concatenate(parts, axis=1) + cb_ref[...][:, :, None]
        u = jnp.maximum(_up(c), 0.0)

    hp = jnp.concatenate([u, jnp.zeros((32, 4, _B), jnp.float32)], axis=1)
    rows = []
    for s0 in range(0, 256, 64):
        w = hp[:, s0:s0 + 68, :].reshape(2176, _B)
        rows.append(dot(m5_ref[...], w))
    c5 = jnp.concatenate(rows, axis=0)
    o_ref[...] = c5[1:251, :] + cb5_ref[0, 0]


def _mk_w1(cw1):
    wt = cw1[:, 0, :].T
    arr = jnp.zeros((18, 32, 16), jnp.float32)
    for d in range(16):
        arr = arr.at[d:d + 3, :, d].set(wt)
    return arr.reshape(18, 512).T


def _mk_stage(cw):
    wt = jnp.transpose(cw, (1, 2, 0))
    arr = jnp.zeros((32, 20, 32, 16), jnp.float32)
    for d in range(16):
        arr = arr.at[:, d:d + 5, :, d].set(wt)
    return arr.reshape(640, 512).T


def _mk_w5(cw5):
    arr = jnp.zeros((32, 68, 64), jnp.float32)
    for d in range(64):
        arr = arr.at[:, d:d + 5, d].set(cw5[0])
    return arr.reshape(2176, 64).T


def kernel(z, a, Wr0, Ws0, b0, Wr1, Ws1, b1, Wr2, Ws2, b2, cw1, cb1, cw2,
           cb2, cw3, cb3, cw4, cb4, cw5, cb5, logvar_x):
    x1 = _gnn_layer(z, a, Wr0, Ws0, b0, False)
    x2 = _gnn_layer(x1, a, Wr1, Ws1, b1, False)
    hT = _gnn_layer(x2, a, Wr2, Ws2, b2, True)

    m1 = _mk_w1(cw1)
    m2 = _mk_stage(cw2)
    m3 = _mk_stage(cw3)
    m4 = _mk_stage(cw4)
    m5 = _mk_w5(cw5)
    full = lambda shape: pl.BlockSpec(shape, lambda i: (0, 0))
    out2d = pl.pallas_call(
        _cnn_body,
        grid=(_N // _B,),
        in_specs=[
            full((512, 18)), full((32, 1)),
            full((512, 640)), full((32, 1)),
            full((512, 640)), full((32, 1)),
            full((512, 640)), full((32, 1)),
            full((64, 2176)), full((1, 1)),
            pl.BlockSpec((16, _B), lambda i: (0, i)),
        ],
        out_specs=pl.BlockSpec((250, _B), lambda i: (0, i)),
        out_shape=jax.ShapeDtypeStruct((250, _N), jnp.float32),
        compiler_params=pltpu.CompilerParams(
            dimension_semantics=("parallel",)),
    )(m1, cb1[:, None], m2, cb2[:, None], m3, cb3[:, None], m4,
      cb4[:, None], m5, cb5[:, None], hT)

    x = out2d.T.reshape(1, _N, 250)
    return x, jnp.exp(0.5 * logvar_x)


# speedup vs baseline: 2.1817x; 1.5048x over previous
import functools

import jax
import jax.numpy as jnp
import numpy as np
from jax.experimental import pallas as pl
from jax.experimental.pallas import tpu as pltpu

_N = 4096
_IB = 512
_JB = 1024
_B = 256


def _split(x):
    hi = x.astype(jnp.bfloat16)
    lo = (x - hi.astype(jnp.float32)).astype(jnp.bfloat16)
    return hi, lo


def _gnn_body(nj, transpose_out, a_ref, xj_ref, xi_ref, wr_ref, ws_ref,
              b_ref, o_ref, acc_ref):
    j = pl.program_id(1)
    d0 = lambda u, v: jax.lax.dot_general(
        u, v, ((((0,), (0,))), ((), ())),
        preferred_element_type=jnp.float32)
    ah, al = _split(a_ref[...])
    xh, xl = _split(xj_ref[...])
    f = xh.shape[1]
    if transpose_out:
        p = d0(jnp.concatenate([xh, xl], axis=1), ah)
        part = p[:f] + p[f:] + d0(xh, al)
    else:
        p = d0(ah, jnp.concatenate([xh, xl], axis=1))
        part = p[:, :f] + p[:, f:] + d0(al, xh)

    @pl.when(j == 0)
    def _():
        acc_ref[...] = part

    @pl.when(j > 0)
    def _():
        acc_ref[...] += part

    @pl.when(j == nj - 1)
    def _():
        agg = acc_ref[...]
        if transpose_out:
            lin = jax.lax.dot_general(
                wr_ref[...], agg, ((((0,), (0,))), ((), ())),
                preferred_element_type=jnp.float32)
            lin += jax.lax.dot_general(
                ws_ref[...], xi_ref[...], ((((0,), (1,))), ((), ())),
                preferred_element_type=jnp.float32)
        else:
            lin = jnp.dot(agg, wr_ref[...],
                          preferred_element_type=jnp.float32)
            lin += jnp.dot(xi_ref[...], ws_ref[...],
                           preferred_element_type=jnp.float32)
        o_ref[...] = jnp.maximum(lin + b_ref[...], 0.0)


def _gnn_layer(x, a, Wr, Ws, b, transpose_out):
    f_in = x.shape[1]
    f_out = Wr.shape[1]
    ni, nj = _N // _IB, _N // _JB
    b2 = b[:, None] if transpose_out else b[None, :]
    if transpose_out:
        out_shape = jax.ShapeDtypeStruct((f_out, _N), jnp.float32)
        out_spec = pl.BlockSpec((f_out, _IB), lambda i, j: (0, i))
        acc_shape = pltpu.VMEM((f_in, _IB), jnp.float32)
    else:
        out_shape = jax.ShapeDtypeStruct((_N, f_out), jnp.float32)
        out_spec = pl.BlockSpec((_IB, f_out), lambda i, j: (i, 0))
        acc_shape = pltpu.VMEM((_IB, f_in), jnp.float32)
    return pl.pallas_call(
        functools.partial(_gnn_body, nj, transpose_out),
        grid=(ni, nj),
        in_specs=[
            pl.BlockSpec((_JB, _IB), lambda i, j: (j, i)),
            pl.BlockSpec((_JB, f_in), lambda i, j: (j, 0)),
            pl.BlockSpec((_IB, f_in), lambda i, j: (i, 0)),
            pl.BlockSpec((f_in, f_out), lambda i, j: (0, 0)),
            pl.BlockSpec((f_in, f_out), lambda i, j: (0, 0)),
            pl.BlockSpec(b2.shape, lambda i, j: (0, 0)),
        ],
        out_specs=out_spec,
        out_shape=out_shape,
        scratch_shapes=[acc_shape],
        compiler_params=pltpu.CompilerParams(
            dimension_semantics=("parallel", "arbitrary")),
    )(a, x, x, Wr, Ws, b2)


def _up2(c):
    prev = jnp.concatenate([c[:, :1], c[:, :-1]], axis=1)
    nxt = jnp.concatenate([c[:, 1:], c[:, -1:]], axis=1)
    e = jnp.maximum(0.75 * c + 0.25 * prev, 0.0)
    o = jnp.maximum(0.75 * c + 0.25 * nxt, 0.0)
    return e, o


def _pad1(x, left, right):
    z = lambda n: jnp.zeros((x.shape[0], n, x.shape[2]), x.dtype)
    pieces = ([z(left)] if left else []) + [x] + ([z(right)] if right else [])
    return jnp.concatenate(pieces, axis=1)


def _cnn_body(m1_ref, cb1_ref, m2_ref, cb2_ref, m3_ref, cb3_ref, m4_ref,
              cb4_ref, m5_ref, cb5_ref, h_ref, o_ref):
    dot = lambda u, v: jax.lax.dot_general(
        u, v, ((((1,), (0,))), ((), ())),
        preferred_element_type=jnp.float32)
    x0 = h_ref[...]
    z1 = jnp.zeros((1, _B), jnp.float32)
    xp = jnp.concatenate([z1, x0, z1], axis=0)
    c = dot(m1_ref[...], xp)
    c = c.reshape(32, 16, _B) + cb1_ref[...][:, :, None]
    e, o = _up2(c)

    for m_ref, cb_ref in ((m2_ref, cb2_ref), (m3_ref, cb3_ref),
                          (m4_ref, cb4_ref)):
        pp = e.shape[1]
        ep, op = _pad1(e, 1, 1), _pad1(o, 1, 1)
        parts = []
        for lb in range(pp // 8):
            s = slice(lb * 8, lb * 8 + 10)
            w = jnp.concatenate([ep[:, s, :], op[:, s, :]],
                                axis=1).reshape(640, _B)
            parts.append(dot(m_ref[...], w).reshape(32, 16, _B))
        c = jnp.concatenate(parts, axis=1) + cb_ref[...][:, :, None]
        e, o = _up2(c)

    ep, op = _pad1(e, 0, 2), _pad1(o, 0, 2)
    rows = []
    for k in range(4):
        s = slice(32 * k, 32 * k + 34)
        w = jnp.concatenate([ep[:, s, :], op[:, s, :]],
                            axis=1).reshape(2176, _B)
        rows.append(dot(m5_ref[...], w))
    c5 = jnp.concatenate(rows, axis=0)
    o_ref[...] = c5[1:251, :] + cb5_ref[0, 0]


def _mk_w1(cw1):
    wt = cw1[:, 0, :].T
    arr = jnp.zeros((18, 32, 16), jnp.float32)
    for d in range(16):
        arr = arr.at[d:d + 3, :, d].set(wt)
    return arr.reshape(18, 512).T


def _mk_stage(cw):
    wt = jnp.transpose(cw, (1, 2, 0))
    ph, m, dl = np.meshgrid(np.arange(2), np.arange(10), np.arange(16),
                            indexing='ij')
    tau = 2 * m + ph - dl
    valid = jnp.asarray((tau >= 0) & (tau < 5))
    g = wt[:, np.clip(tau, 0, 4), :]
    g = jnp.where(valid[None, :, :, :, None], g, 0.0)
    g = jnp.transpose(g, (0, 1, 2, 4, 3))
    return g.reshape(640, 512).T


def _mk_w5(cw5):
    wt = cw5[0]
    ph, m, dl = np.meshgrid(np.arange(2), np.arange(34), np.arange(64),
                            indexing='ij')
    tau = 2 * m + ph - dl
    valid = jnp.asarray((tau >= 0) & (tau < 5))
    g = wt[:, np.clip(tau, 0, 4)]
    g = jnp.where(valid[None], g, 0.0)
    return g.reshape(2176, 64).T


def kernel(z, a, Wr0, Ws0, b0, Wr1, Ws1, b1, Wr2, Ws2, b2, cw1, cb1, cw2,
           cb2, cw3, cb3, cw4, cb4, cw5, cb5, logvar_x):
    x1 = _gnn_layer(z, a, Wr0, Ws0, b0, False)
    x2 = _gnn_layer(x1, a, Wr1, Ws1, b1, False)
    hT = _gnn_layer(x2, a, Wr2, Ws2, b2, True)

    m1 = _mk_w1(cw1)
    m2 = _mk_stage(cw2)
    m3 = _mk_stage(cw3)
    m4 = _mk_stage(cw4)
    m5 = _mk_w5(cw5)
    full = lambda shape: pl.BlockSpec(shape, lambda i: (0, 0))
    out2d = pl.pallas_call(
        _cnn_body,
        grid=(_N // _B,),
        in_specs=[
            full((512, 18)), full((32, 1)),
            full((512, 640)), full((32, 1)),
            full((512, 640)), full((32, 1)),
            full((512, 640)), full((32, 1)),
            full((64, 2176)), full((1, 1)),
            pl.BlockSpec((16, _B), lambda i: (0, i)),
        ],
        out_specs=pl.BlockSpec((250, _B), lambda i: (0, i)),
        out_shape=jax.ShapeDtypeStruct((250, _N), jnp.float32),
        compiler_params=pltpu.CompilerParams(
            dimension_semantics=("parallel",)),
    )(m1, cb1[:, None], m2, cb2[:, None], m3, cb3[:, None], m4,
      cb4[:, None], m5, cb5[:, None], hT)

    x = out2d.T.reshape(1, _N, 250)
    return x, jnp.exp(0.5 * logvar_x)
